# Initial kernel scaffold; baseline (speedup 1.0000x reference)
import jax
import jax.numpy as jnp
from jax.experimental import pallas as pl


def kernel(x, edge_index, etypes, Wt, bt, W_ih, W_hh, b_ih, b_hh, conv1_w, conv1_b, conv2_w, conv2_b, convc1_w, convc1_b, convc2_w, convc2_b, mlp_y_w, mlp_y_b, mlp_z_w, mlp_z_b):
    raise NotImplementedError("write your pallas kernel here")


# speedup vs baseline: 8.9982x; 8.9982x over previous
import functools

import jax
import jax.numpy as jnp
from jax import lax
from jax.experimental import pallas as pl
from jax.experimental.pallas import tpu as pltpu
from jax.experimental.pallas import tpu_sc as plsc

N = 10002
E = 320064
D = 128
NT = 4
STEPS = 6

NTILES = 32
NPAD = 10112
RPT = NPAD // 16
CH = 128
EPT = 10112
NCHUNK = EPT // CH
EPAD = EPT * NTILES

_sc_mesh = plsc.VectorSubcoreMesh(core_axis_name="c", subcore_axis_name="s")


@functools.partial(
    pl.kernel,
    out_type=(
        jax.ShapeDtypeStruct((NPAD, D), jnp.float32),
        jax.ShapeDtypeStruct((NPAD, D), jnp.float32),
    ),
    mesh=_sc_mesh,
    scratch_types=[
        pltpu.VMEM((CH,), jnp.int32),
        pltpu.VMEM((CH,), jnp.int32),
        pltpu.VMEM((CH, D), jnp.float32),
        pltpu.VMEM_SHARED((NPAD, D), jnp.float32),
        pltpu.SemaphoreType.DMA,
    ],
)
def _edge_pass(lin_hbm, gidx_hbm, dst_hbm, zeros_hbm,
               out0, out1, idx_v, dst_v, rows_v, acc_sh, sem):
    c = lax.axis_index("c")
    s = lax.axis_index("s")
    lo = s * RPT
    pltpu.sync_copy(zeros_hbm.at[pl.ds(lo, RPT)], acc_sh.at[pl.ds(lo, RPT)])
    plsc.subcore_barrier()

    wid = c * 16 + s
    base = wid * EPT

    def body(g, carry):
        e0 = pl.multiple_of(base + g * CH, CH)
        pltpu.sync_copy(gidx_hbm.at[pl.ds(e0, CH)], idx_v)
        pltpu.sync_copy(dst_hbm.at[pl.ds(e0, CH)], dst_v)
        pltpu.async_copy(lin_hbm.at[idx_v], rows_v, sem).wait()
        pltpu.sync_copy(rows_v, acc_sh.at[dst_v], add=True)
        return carry

    lax.fori_loop(0, NCHUNK, body, 0)
    plsc.subcore_barrier()

    @pl.when(c == 0)
    def _():
        pltpu.sync_copy(acc_sh.at[pl.ds(lo, RPT)], out0.at[pl.ds(lo, RPT)])

    @pl.when(c == 1)
    def _():
        pltpu.sync_copy(acc_sh.at[pl.ds(lo, RPT)], out1.at[pl.ds(lo, RPT)])


BN = 2048


def _lin0_body(h_ref, wcat_ref, bcat_ref, lin_ref):
    lin_ref[...] = (
        jnp.dot(h_ref[...], wcat_ref[...], preferred_element_type=jnp.float32)
        + bcat_ref[...]
    )


def _gru_body(h_ref, p0_ref, p1_ref, wih_ref, whh_ref, bih_ref, bhh_ref,
              wcat_ref, bcat_ref, hout_ref, lin_ref):
    h = h_ref[...]
    a = p0_ref[...] + p1_ref[...]
    gi = jnp.dot(a, wih_ref[...], preferred_element_type=jnp.float32) + bih_ref[...]
    gh = jnp.dot(h, whh_ref[...], preferred_element_type=jnp.float32) + bhh_ref[...]
    r = jax.nn.sigmoid(gi[:, :D] + gh[:, :D])
    z = jax.nn.sigmoid(gi[:, D:2 * D] + gh[:, D:2 * D])
    n = jnp.tanh(gi[:, 2 * D:] + r * gh[:, 2 * D:])
    hn = (1.0 - z) * n + z * h
    hout_ref[...] = hn
    if lin_ref is not None:
        lin_ref[...] = (
            jnp.dot(hn, wcat_ref[...], preferred_element_type=jnp.float32)
            + bcat_ref[...]
        )


def _gru_last_body(h_ref, p0_ref, p1_ref, wih_ref, whh_ref, bih_ref, bhh_ref,
                   hout_ref):
    _gru_body(h_ref, p0_ref, p1_ref, wih_ref, whh_ref, bih_ref, bhh_ref,
              None, None, hout_ref, None)


def _ypath_body(h_ref, w0, w1, w2, b1, w3, b3, wy, by, yv_ref):
    X = h_ref[...]
    Y0 = (jnp.dot(X[0:10000], w0[...], preferred_element_type=jnp.float32)
          + jnp.dot(X[1:10001], w1[...], preferred_element_type=jnp.float32)
          + jnp.dot(X[2:10002], w2[...], preferred_element_type=jnp.float32)
          + b1[...])
    Y0 = jnp.maximum(Y0, 0.0)
    Ev = Y0.reshape(5000, 2, D)
    A = jnp.maximum(Ev[:, 0, :], Ev[:, 1, :])
    Y1 = jnp.maximum(A[0:4999], Ev[1:5000, 0, :])
    Y1 = jnp.maximum(jnp.dot(Y1, w3[...], preferred_element_type=jnp.float32)
                     + b3[...], 0.0)
    Yp = Y1[0:4998].reshape(2499, 2, D)
    Y2 = jnp.maximum(Yp[:, 0, :], Yp[:, 1, :])
    yv_ref[...] = jnp.dot(Y2, wy[...], preferred_element_type=jnp.float32) + by[...]


def _zpath_body(h_ref, x_ref, cw0h, cw0x, cw1h, cw1x, cw2h, cw2x, cb1,
                dw0, dw1, db2, wz, bz, yv_ref, out_ref):
    H = h_ref[...]
    X = x_ref[...]
    Z0 = (jnp.dot(H[0:10000], cw0h[...], preferred_element_type=jnp.float32)
          + jnp.dot(X[0:10000], cw0x[...], preferred_element_type=jnp.float32)
          + jnp.dot(H[1:10001], cw1h[...], preferred_element_type=jnp.float32)
          + jnp.dot(X[1:10001], cw1x[...], preferred_element_type=jnp.float32)
          + jnp.dot(H[2:10002], cw2h[...], preferred_element_type=jnp.float32)
          + jnp.dot(X[2:10002], cw2x[...], preferred_element_type=jnp.float32)
          + cb1[...])
    Z0 = jnp.maximum(Z0, 0.0)
    Ev = Z0.reshape(5000, 2, 2 * D)
    A = jnp.maximum(Ev[:, 0, :], Ev[:, 1, :])
    Z1 = jnp.maximum(A[0:4999], Ev[1:5000, 0, :])
    Z2 = (jnp.dot(Z1[0:4998], dw0[...], preferred_element_type=jnp.float32)
          + jnp.dot(Z1[1:4999], dw1[...], preferred_element_type=jnp.float32)
          + db2[...])
    Z2 = jnp.maximum(Z2, 0.0)
    Zp = Z2.reshape(2499, 2, 2 * D)
    Z2p = jnp.maximum(Zp[:, 0, :], Zp[:, 1, :])
    zv = jnp.dot(Z2p, wz[...], preferred_element_type=jnp.float32) + bz[...]
    prod = zv * yv_ref[...]
    m = jnp.sum(prod) / jnp.float32(2499.0)
    out_ref[...] = jax.nn.sigmoid(m).reshape(1, 1)


def _row_blocked(shape):
    return pl.BlockSpec((BN, shape[1]), lambda i: (i, 0))


def _full(shape):
    return pl.BlockSpec(shape, lambda *_: (0,) * len(shape))


def kernel(x, edge_index, etypes, Wt, bt, W_ih, W_hh, b_ih, b_hh,
           conv1_w, conv1_b, conv2_w, conv2_b, convc1_w, convc1_b,
           convc2_w, convc2_b, mlp_y_w, mlp_y_b, mlp_z_w, mlp_z_b):
    f32 = jnp.float32
    h0 = x[0]

    wcat = jnp.transpose(Wt, (2, 0, 1)).reshape(D, NT * D)
    bcat = bt.reshape(1, NT * D)
    wih = W_ih.T
    whh = W_hh.T
    bih = b_ih.reshape(1, 3 * D)
    bhh = b_hh.reshape(1, 3 * D)

    src = edge_index[0]
    dst = edge_index[1]
    gidx = src * NT + etypes
    gidx_p = jnp.concatenate([gidx, jnp.zeros((EPAD - E,), jnp.int32)])
    dst_p = jnp.concatenate([dst, jnp.full((EPAD - E,), N, jnp.int32)])
    zeros_acc = jnp.zeros((NPAD, D), f32)

    grid_n = (N + BN - 1) // BN

    lin = pl.pallas_call(
        _lin0_body,
        grid=(grid_n,),
        in_specs=[_row_blocked((N, D)), _full((D, NT * D)), _full((1, NT * D))],
        out_specs=_row_blocked((N, NT * D)),
        out_shape=jax.ShapeDtypeStruct((N, NT * D), f32),
    )(h0, wcat, bcat)

    h = h0
    for step in range(STEPS):
        lin4 = lin.reshape(NT * N, D)
        p0, p1 = _edge_pass(lin4, gidx_p, dst_p, zeros_acc)
        p0 = p0[:N]
        p1 = p1[:N]
        if step < STEPS - 1:
            h, lin = pl.pallas_call(
                _gru_body,
                grid=(grid_n,),
                in_specs=[_row_blocked((N, D)), _row_blocked((N, D)),
                          _row_blocked((N, D)), _full((D, 3 * D)),
                          _full((D, 3 * D)), _full((1, 3 * D)),
                          _full((1, 3 * D)), _full((D, NT * D)),
                          _full((1, NT * D))],
                out_specs=[_row_blocked((N, D)), _row_blocked((N, NT * D))],
                out_shape=[jax.ShapeDtypeStruct((N, D), f32),
                           jax.ShapeDtypeStruct((N, NT * D), f32)],
            )(h, p0, p1, wih, whh, bih, bhh, wcat, bcat)
        else:
            h = pl.pallas_call(
                _gru_last_body,
                grid=(grid_n,),
                in_specs=[_row_blocked((N, D)), _row_blocked((N, D)),
                          _row_blocked((N, D)), _full((D, 3 * D)),
                          _full((D, 3 * D)), _full((1, 3 * D)),
                          _full((1, 3 * D))],
                out_specs=_row_blocked((N, D)),
                out_shape=jax.ShapeDtypeStruct((N, D), f32),
            )(h, p0, p1, wih, whh, bih, bhh)

    w0 = conv1_w[:, :, 0].T
    w1 = conv1_w[:, :, 1].T
    w2 = conv1_w[:, :, 2].T
    b1 = conv1_b.reshape(1, D)
    w3 = conv2_w[:, :, 0].T
    b3 = conv2_b.reshape(1, D)
    wy = mlp_y_w.T
    by = mlp_y_b.reshape(1, 1)

    yv = pl.pallas_call(
        _ypath_body,
        in_specs=[_full((N, D))] + [_full(s) for s in
                  [(D, D), (D, D), (D, D), (1, D), (D, D), (1, D), (D, 1), (1, 1)]],
        out_specs=_full((2499, 1)),
        out_shape=jax.ShapeDtypeStruct((2499, 1), f32),
    )(h, w0, w1, w2, b1, w3, b3, wy, by)

    C2 = 2 * D
    cw0 = convc1_w[:, :, 0].T
    cw1 = convc1_w[:, :, 1].T
    cw2 = convc1_w[:, :, 2].T
    cb1 = convc1_b.reshape(1, C2)
    dw0 = convc2_w[:, :, 0].T
    dw1 = convc2_w[:, :, 1].T
    db2 = convc2_b.reshape(1, C2)
    wz = mlp_z_w.T
    bz = mlp_z_b.reshape(1, 1)

    out = pl.pallas_call(
        _zpath_body,
        in_specs=[_full((N, D)), _full((N, D)),
                  _full((D, C2)), _full((D, C2)), _full((D, C2)),
                  _full((D, C2)), _full((D, C2)), _full((D, C2)),
                  _full((1, C2)), _full((C2, C2)), _full((C2, C2)),
                  _full((1, C2)), _full((C2, 1)), _full((1, 1)),
                  _full((2499, 1))],
        out_specs=_full((1, 1)),
        out_shape=jax.ShapeDtypeStruct((1, 1), f32),
    )(h, h0, cw0[:D], cw0[D:], cw1[:D], cw1[D:], cw2[:D], cw2[D:],
      cb1, dw0, dw1, db2, wz, bz, yv)

    return out.reshape(1)
